# baseline (speedup 1.0000x reference)
import jax
import jax.numpy as jnp
from jax import lax
from jax.experimental import pallas as pl
from jax.experimental.pallas import tpu as pltpu
from jax.experimental.pallas import tpu_sc as plsc

N_NODES_K = 50000
D_K = 256
G_K = 64
EPS_K = 1e-6

ROWS_BLK = 2000

N_SC_K = 8000
N_TC_K = N_NODES_K - N_SC_K
TC_BLK0 = N_SC_K // ROWS_BLK
N_TC_BLKS = N_TC_K // ROWS_BLK
N_BLKS = N_NODES_K // ROWS_BLK

CHUNK_K = 80
N_CHUNKS_K = N_SC_K // CHUNK_K
NW_K = 32
BASE_CHUNKS_K = N_CHUNKS_K // NW_K
EXTRA_K = N_CHUNKS_K - BASE_CHUNKS_K * NW_K

NJ_K = D_K // 16


def _sc_sums_body(x_hbm, seg_hbm, s1_out, s2_out, cnt_out,
                  xbuf0, xbuf1, idxbuf0, idxbuf1, acc1, acc2, acc_c, dsem):
    c = lax.axis_index("c")
    s = lax.axis_index("s")
    wid = c * 16 + s

    zeros16 = jnp.zeros((16,), jnp.float32)
    ones16 = jnp.ones((16,), jnp.float32)

    def zero_body(g, carry):
        for j in range(NJ_K):
            acc1[g, pl.ds(j * 16, 16)] = zeros16
            acc2[g, pl.ds(j * 16, 16)] = zeros16
        acc_c[g, :] = zeros16
        return carry

    lax.fori_loop(0, G_K, zero_body, 0)

    n_ch = BASE_CHUNKS_K + jnp.where(wid < EXTRA_K, 1, 0)
    c0 = wid * BASE_CHUNKS_K + jnp.minimum(wid, EXTRA_K)

    def process(xbuf, idxbuf):
        s_first = jnp.max(plsc.load_gather(
            idxbuf, [jnp.zeros((16,), jnp.int32)]))
        s_last = jnp.max(plsc.load_gather(
            idxbuf, [jnp.full((16,), CHUNK_K - 1, jnp.int32)]))

        @pl.when(s_first == s_last)
        def _fast():
            def row_body(r2, accs):
                s1s, s2s = accs
                r = r2 * 2
                vs = [xbuf[r + rr, pl.ds(j * 16, 16)]
                      for rr in range(2) for j in range(NJ_K)]
                n1 = []
                n2 = []
                for j in range(NJ_K):
                    a, b = vs[j], vs[NJ_K + j]
                    n1.append(s1s[j] + (a + b))
                    n2.append(s2s[j] + (a * a + b * b))
                return (tuple(n1), tuple(n2))

            init = (tuple(zeros16 for _ in range(NJ_K)),
                    tuple(zeros16 for _ in range(NJ_K)))
            s1s, s2s = lax.fori_loop(0, CHUNK_K // 2, row_body, init)
            for j in range(NJ_K):
                sl = pl.ds(j * 16, 16)
                acc1[s_first, sl] = acc1[s_first, sl] + s1s[j]
                acc2[s_first, sl] = acc2[s_first, sl] + s2s[j]
            acc_c[s_first, :] = acc_c[s_first, :] + float(CHUNK_K) * ones16

        @pl.when(s_first != s_last)
        def _slow():
            def row_body(r, carry2):
                sv = jnp.max(plsc.load_gather(
                    idxbuf, [jnp.full((16,), r, jnp.int32)]))
                for j in range(NJ_K):
                    sl = pl.ds(j * 16, 16)
                    v = xbuf[r, sl]
                    acc1[sv, sl] = acc1[sv, sl] + v
                    acc2[sv, sl] = acc2[sv, sl] + v * v
                acc_c[sv, :] = acc_c[sv, :] + ones16
                return carry2

            lax.fori_loop(0, CHUNK_K, row_body, 0)

    pltpu.sync_copy(x_hbm.at[pl.ds(c0 * CHUNK_K, CHUNK_K)], xbuf0)
    pltpu.sync_copy(seg_hbm.at[pl.ds(c0 * CHUNK_K, CHUNK_K)], idxbuf0)

    def chunk_body(ch, carry):
        par = lax.rem(ch, 2)
        have_next = ch + 1 < n_ch
        base_next = (c0 + ch + 1) * CHUNK_K

        @pl.when(jnp.logical_and(have_next, par == 1))
        def _pf0():
            pltpu.async_copy(x_hbm.at[pl.ds(base_next, CHUNK_K)], xbuf0, dsem)
            pltpu.async_copy(seg_hbm.at[pl.ds(base_next, CHUNK_K)], idxbuf0, dsem)

        @pl.when(jnp.logical_and(have_next, par == 0))
        def _pf1():
            pltpu.async_copy(x_hbm.at[pl.ds(base_next, CHUNK_K)], xbuf1, dsem)
            pltpu.async_copy(seg_hbm.at[pl.ds(base_next, CHUNK_K)], idxbuf1, dsem)

        @pl.when(par == 0)
        def _p0():
            process(xbuf0, idxbuf0)

        @pl.when(par == 1)
        def _p1():
            process(xbuf1, idxbuf1)

        @pl.when(have_next)
        def _drain():
            pltpu.make_async_copy(
                x_hbm.at[pl.ds(base_next, CHUNK_K)], xbuf0, dsem).wait()
            pltpu.make_async_copy(
                seg_hbm.at[pl.ds(base_next, CHUNK_K)], idxbuf0, dsem).wait()

        return carry

    lax.fori_loop(0, n_ch, chunk_body, 0)

    pltpu.sync_copy(acc1, s1_out.at[wid])
    pltpu.sync_copy(acc2, s2_out.at[wid])
    pltpu.sync_copy(acc_c, cnt_out.at[wid])


def _sc_segment_sums(x, seg):
    mesh = plsc.VectorSubcoreMesh(core_axis_name="c", subcore_axis_name="s")
    fn = pl.kernel(
        _sc_sums_body,
        mesh=mesh,
        compiler_params=pltpu.CompilerParams(needs_layout_passes=False),
        out_type=[
            jax.ShapeDtypeStruct((NW_K, G_K, D_K), jnp.float32),
            jax.ShapeDtypeStruct((NW_K, G_K, D_K), jnp.float32),
            jax.ShapeDtypeStruct((NW_K, G_K, 16), jnp.float32),
        ],
        scratch_types=[
            pltpu.VMEM((CHUNK_K, D_K), jnp.float32),
            pltpu.VMEM((CHUNK_K, D_K), jnp.float32),
            pltpu.VMEM((CHUNK_K,), jnp.int32),
            pltpu.VMEM((CHUNK_K,), jnp.int32),
            pltpu.VMEM((G_K, D_K), jnp.float32),
            pltpu.VMEM((G_K, D_K), jnp.float32),
            pltpu.VMEM((G_K, 16), jnp.float32),
            pltpu.SemaphoreType.DMA,
        ],
    )
    return fn(x, seg)


def _tc_sums_body(x_ref, seg_ref, s1_ref, s2_ref, cnt_ref):
    i = pl.program_id(0)

    @pl.when(i == 0)
    def _init():
        s1_ref[...] = jnp.zeros_like(s1_ref)
        s2_ref[...] = jnp.zeros_like(s2_ref)
        cnt_ref[...] = jnp.zeros_like(cnt_ref)

    x = x_ref[...]
    seg = seg_ref[pl.ds(i + TC_BLK0, 1), :][0]
    oh_t = (lax.broadcasted_iota(jnp.int32, (G_K, ROWS_BLK), 0) == seg[None, :]).astype(jnp.float32)
    s1_ref[...] += jnp.dot(oh_t, x, preferred_element_type=jnp.float32)
    s2_ref[...] += jnp.dot(oh_t, x * x, preferred_element_type=jnp.float32)
    cnt_ref[...] += jnp.broadcast_to(jnp.sum(oh_t, axis=1, keepdims=True), (G_K, 128))


def _stats_body(s1p_ref, s2p_ref, cntp_ref, s1t_ref, s2t_ref, cntt_ref, b_ref):
    s1 = jnp.sum(s1p_ref[...], axis=0) + s1t_ref[...]
    s2 = jnp.sum(s2p_ref[...], axis=0) + s2t_ref[...]
    cnt = jnp.sum(cntp_ref[...], axis=0)[:, 0:1] + cntt_ref[:, 0:1]
    cnt = jnp.maximum(cnt, 1.0)
    inv = 1.0 / cnt
    mu = s1 * inv
    var = s2 * inv - mu * mu
    b_ref[...] = mu * lax.rsqrt(var + EPS_K)


def _apply_body(x_ref, seg_ref, b_ref, out_ref):
    seg = seg_ref[pl.ds(pl.program_id(0), 1), :][0]
    oh = (seg[:, None] == lax.broadcasted_iota(jnp.int32, (ROWS_BLK, G_K), 1)).astype(jnp.float32)
    out_ref[...] = x_ref[...] - jnp.dot(oh, b_ref[...], preferred_element_type=jnp.float32)


def kernel(x, segment_ids):
    seg = segment_ids.astype(jnp.int32)

    s1p, s2p, cntp = _sc_segment_sums(x, seg)
    seg2d = seg.reshape(N_BLKS, ROWS_BLK)

    s1t, s2t, cntt = pl.pallas_call(
        _tc_sums_body,
        grid=(N_TC_BLKS,),
        in_specs=[
            pl.BlockSpec((ROWS_BLK, D_K), lambda i: (i + TC_BLK0, 0)),
            pl.BlockSpec((N_BLKS, ROWS_BLK), lambda i: (0, 0)),
        ],
        out_specs=[
            pl.BlockSpec((G_K, D_K), lambda i: (0, 0)),
            pl.BlockSpec((G_K, D_K), lambda i: (0, 0)),
            pl.BlockSpec((G_K, 128), lambda i: (0, 0)),
        ],
        out_shape=[
            jax.ShapeDtypeStruct((G_K, D_K), jnp.float32),
            jax.ShapeDtypeStruct((G_K, D_K), jnp.float32),
            jax.ShapeDtypeStruct((G_K, 128), jnp.float32),
        ],
    )(x, seg2d)

    b = pl.pallas_call(
        _stats_body,
        out_shape=jax.ShapeDtypeStruct((G_K, D_K), jnp.float32),
    )(s1p, s2p, cntp, s1t, s2t, cntt)

    out = pl.pallas_call(
        _apply_body,
        grid=(N_BLKS,),
        in_specs=[
            pl.BlockSpec((ROWS_BLK, D_K), lambda i: (i, 0)),
            pl.BlockSpec((N_BLKS, ROWS_BLK), lambda i: (0, 0)),
            pl.BlockSpec((G_K, D_K), lambda i: (0, 0)),
        ],
        out_specs=pl.BlockSpec((ROWS_BLK, D_K), lambda i: (i, 0)),
        out_shape=jax.ShapeDtypeStruct((N_NODES_K, D_K), jnp.float32),
    )(x, seg2d, b)

    return out


# speedup vs baseline: 1.0645x; 1.0645x over previous
import jax
import jax.numpy as jnp
from jax import lax
from jax.experimental import pallas as pl
from jax.experimental.pallas import tpu as pltpu
from jax.experimental.pallas import tpu_sc as plsc

N_NODES_K = 50000
D_K = 256
G_K = 64
EPS_K = 1e-6

ROWS_BLK = 5000

N_SC_K = 10000
N_TC_K = N_NODES_K - N_SC_K
TC_BLK0 = N_SC_K // ROWS_BLK
N_TC_BLKS = N_TC_K // ROWS_BLK
N_BLKS = N_NODES_K // ROWS_BLK

CHUNK_K = 80
N_CHUNKS_K = N_SC_K // CHUNK_K
NW_K = 32
BASE_CHUNKS_K = N_CHUNKS_K // NW_K
EXTRA_K = N_CHUNKS_K - BASE_CHUNKS_K * NW_K

NJ_K = D_K // 16


def _sc_sums_body(x_hbm, seg_hbm, s1_out, s2_out, cnt_out,
                  xbuf0, xbuf1, idxbuf0, idxbuf1, acc1, acc2, acc_c, dsem):
    c = lax.axis_index("c")
    s = lax.axis_index("s")
    wid = c * 16 + s

    zeros16 = jnp.zeros((16,), jnp.float32)
    ones16 = jnp.ones((16,), jnp.float32)

    def zero_body(g, carry):
        for j in range(NJ_K):
            acc1[g, pl.ds(j * 16, 16)] = zeros16
            acc2[g, pl.ds(j * 16, 16)] = zeros16
        acc_c[g, :] = zeros16
        return carry

    lax.fori_loop(0, G_K, zero_body, 0)

    n_ch = BASE_CHUNKS_K + jnp.where(wid < EXTRA_K, 1, 0)
    c0 = wid * BASE_CHUNKS_K + jnp.minimum(wid, EXTRA_K)

    def process(xbuf, idxbuf):
        s_first = jnp.max(plsc.load_gather(
            idxbuf, [jnp.zeros((16,), jnp.int32)]))
        s_last = jnp.max(plsc.load_gather(
            idxbuf, [jnp.full((16,), CHUNK_K - 1, jnp.int32)]))

        @pl.when(s_first == s_last)
        def _fast():
            def row_body(r2, accs):
                s1s, s2s = accs
                r = r2 * 2
                vs = [xbuf[r + rr, pl.ds(j * 16, 16)]
                      for rr in range(2) for j in range(NJ_K)]
                n1 = []
                n2 = []
                for j in range(NJ_K):
                    a, b = vs[j], vs[NJ_K + j]
                    n1.append(s1s[j] + (a + b))
                    n2.append(s2s[j] + (a * a + b * b))
                return (tuple(n1), tuple(n2))

            init = (tuple(zeros16 for _ in range(NJ_K)),
                    tuple(zeros16 for _ in range(NJ_K)))
            s1s, s2s = lax.fori_loop(0, CHUNK_K // 2, row_body, init)
            for j in range(NJ_K):
                sl = pl.ds(j * 16, 16)
                acc1[s_first, sl] = acc1[s_first, sl] + s1s[j]
                acc2[s_first, sl] = acc2[s_first, sl] + s2s[j]
            acc_c[s_first, :] = acc_c[s_first, :] + float(CHUNK_K) * ones16

        @pl.when(s_first != s_last)
        def _slow():
            def row_body(r, carry2):
                sv = jnp.max(plsc.load_gather(
                    idxbuf, [jnp.full((16,), r, jnp.int32)]))
                for j in range(NJ_K):
                    sl = pl.ds(j * 16, 16)
                    v = xbuf[r, sl]
                    acc1[sv, sl] = acc1[sv, sl] + v
                    acc2[sv, sl] = acc2[sv, sl] + v * v
                acc_c[sv, :] = acc_c[sv, :] + ones16
                return carry2

            lax.fori_loop(0, CHUNK_K, row_body, 0)

    pltpu.sync_copy(x_hbm.at[pl.ds(c0 * CHUNK_K, CHUNK_K)], xbuf0)
    pltpu.sync_copy(seg_hbm.at[pl.ds(c0 * CHUNK_K, CHUNK_K)], idxbuf0)

    def chunk_body(ch, carry):
        par = lax.rem(ch, 2)
        have_next = ch + 1 < n_ch
        base_next = (c0 + ch + 1) * CHUNK_K

        @pl.when(jnp.logical_and(have_next, par == 1))
        def _pf0():
            pltpu.async_copy(x_hbm.at[pl.ds(base_next, CHUNK_K)], xbuf0, dsem)
            pltpu.async_copy(seg_hbm.at[pl.ds(base_next, CHUNK_K)], idxbuf0, dsem)

        @pl.when(jnp.logical_and(have_next, par == 0))
        def _pf1():
            pltpu.async_copy(x_hbm.at[pl.ds(base_next, CHUNK_K)], xbuf1, dsem)
            pltpu.async_copy(seg_hbm.at[pl.ds(base_next, CHUNK_K)], idxbuf1, dsem)

        @pl.when(par == 0)
        def _p0():
            process(xbuf0, idxbuf0)

        @pl.when(par == 1)
        def _p1():
            process(xbuf1, idxbuf1)

        @pl.when(have_next)
        def _drain():
            pltpu.make_async_copy(
                x_hbm.at[pl.ds(base_next, CHUNK_K)], xbuf0, dsem).wait()
            pltpu.make_async_copy(
                seg_hbm.at[pl.ds(base_next, CHUNK_K)], idxbuf0, dsem).wait()

        return carry

    lax.fori_loop(0, n_ch, chunk_body, 0)

    pltpu.sync_copy(acc1, s1_out.at[wid])
    pltpu.sync_copy(acc2, s2_out.at[wid])
    pltpu.sync_copy(acc_c, cnt_out.at[wid])


def _sc_segment_sums(x, seg):
    mesh = plsc.VectorSubcoreMesh(core_axis_name="c", subcore_axis_name="s")
    fn = pl.kernel(
        _sc_sums_body,
        mesh=mesh,
        compiler_params=pltpu.CompilerParams(needs_layout_passes=False),
        out_type=[
            jax.ShapeDtypeStruct((NW_K, G_K, D_K), jnp.float32),
            jax.ShapeDtypeStruct((NW_K, G_K, D_K), jnp.float32),
            jax.ShapeDtypeStruct((NW_K, G_K, 16), jnp.float32),
        ],
        scratch_types=[
            pltpu.VMEM((CHUNK_K, D_K), jnp.float32),
            pltpu.VMEM((CHUNK_K, D_K), jnp.float32),
            pltpu.VMEM((CHUNK_K,), jnp.int32),
            pltpu.VMEM((CHUNK_K,), jnp.int32),
            pltpu.VMEM((G_K, D_K), jnp.float32),
            pltpu.VMEM((G_K, D_K), jnp.float32),
            pltpu.VMEM((G_K, 16), jnp.float32),
            pltpu.SemaphoreType.DMA,
        ],
    )
    return fn(x, seg)


def _tc_sums_body(x_ref, seg_ref, s1_ref, s2_ref, cnt_ref):
    i = pl.program_id(0)

    @pl.when(i == 0)
    def _init():
        s1_ref[...] = jnp.zeros_like(s1_ref)
        s2_ref[...] = jnp.zeros_like(s2_ref)
        cnt_ref[...] = jnp.zeros_like(cnt_ref)

    x = x_ref[...]
    seg = seg_ref[pl.ds(i + TC_BLK0, 1), :][0]
    oh_t = (lax.broadcasted_iota(jnp.int32, (G_K, ROWS_BLK), 0) == seg[None, :]).astype(jnp.float32)
    s1_ref[...] += jnp.dot(oh_t, x, preferred_element_type=jnp.float32)
    s2_ref[...] += jnp.dot(oh_t, x * x, preferred_element_type=jnp.float32)
    cnt_ref[...] += jnp.broadcast_to(jnp.sum(oh_t, axis=1, keepdims=True), (G_K, 128))


def _stats_body(s1p_ref, s2p_ref, cntp_ref, s1t_ref, s2t_ref, cntt_ref, b_ref):
    s1 = jnp.sum(s1p_ref[...], axis=0) + s1t_ref[...]
    s2 = jnp.sum(s2p_ref[...], axis=0) + s2t_ref[...]
    cnt = jnp.sum(cntp_ref[...], axis=0)[:, 0:1] + cntt_ref[:, 0:1]
    cnt = jnp.maximum(cnt, 1.0)
    inv = 1.0 / cnt
    mu = s1 * inv
    var = s2 * inv - mu * mu
    b_ref[...] = mu * lax.rsqrt(var + EPS_K)


def _apply_body(x_ref, seg_ref, b_ref, out_ref):
    seg = seg_ref[pl.ds(pl.program_id(0), 1), :][0]
    oh = (seg[:, None] == lax.broadcasted_iota(jnp.int32, (ROWS_BLK, G_K), 1)).astype(jnp.float32)
    out_ref[...] = x_ref[...] - jnp.dot(oh, b_ref[...], preferred_element_type=jnp.float32)


def kernel(x, segment_ids):
    seg = segment_ids.astype(jnp.int32)

    s1p, s2p, cntp = _sc_segment_sums(x, seg)
    seg2d = seg.reshape(N_BLKS, ROWS_BLK)

    s1t, s2t, cntt = pl.pallas_call(
        _tc_sums_body,
        grid=(N_TC_BLKS,),
        in_specs=[
            pl.BlockSpec((ROWS_BLK, D_K), lambda i: (i + TC_BLK0, 0)),
            pl.BlockSpec((N_BLKS, ROWS_BLK), lambda i: (0, 0)),
        ],
        out_specs=[
            pl.BlockSpec((G_K, D_K), lambda i: (0, 0)),
            pl.BlockSpec((G_K, D_K), lambda i: (0, 0)),
            pl.BlockSpec((G_K, 128), lambda i: (0, 0)),
        ],
        out_shape=[
            jax.ShapeDtypeStruct((G_K, D_K), jnp.float32),
            jax.ShapeDtypeStruct((G_K, D_K), jnp.float32),
            jax.ShapeDtypeStruct((G_K, 128), jnp.float32),
        ],
    )(x, seg2d)

    b = pl.pallas_call(
        _stats_body,
        out_shape=jax.ShapeDtypeStruct((G_K, D_K), jnp.float32),
    )(s1p, s2p, cntp, s1t, s2t, cntt)

    out = pl.pallas_call(
        _apply_body,
        grid=(N_BLKS,),
        in_specs=[
            pl.BlockSpec((ROWS_BLK, D_K), lambda i: (i, 0)),
            pl.BlockSpec((N_BLKS, ROWS_BLK), lambda i: (0, 0)),
            pl.BlockSpec((G_K, D_K), lambda i: (0, 0)),
        ],
        out_specs=pl.BlockSpec((ROWS_BLK, D_K), lambda i: (i, 0)),
        out_shape=jax.ShapeDtypeStruct((N_NODES_K, D_K), jnp.float32),
    )(x, seg2d, b)

    return out


# speedup vs baseline: 1.0747x; 1.0096x over previous
import jax
import jax.numpy as jnp
from jax import lax
from jax.experimental import pallas as pl
from jax.experimental.pallas import tpu as pltpu
from jax.experimental.pallas import tpu_sc as plsc

N_NODES_K = 50000
D_K = 256
G_K = 64
EPS_K = 1e-6

ROWS_BLK = 10000

N_SC_K = 10000
N_TC_K = N_NODES_K - N_SC_K
TC_BLK0 = N_SC_K // ROWS_BLK
N_TC_BLKS = N_TC_K // ROWS_BLK
N_BLKS = N_NODES_K // ROWS_BLK

CHUNK_K = 80
N_CHUNKS_K = N_SC_K // CHUNK_K
NW_K = 32
BASE_CHUNKS_K = N_CHUNKS_K // NW_K
EXTRA_K = N_CHUNKS_K - BASE_CHUNKS_K * NW_K

NJ_K = D_K // 16


def _sc_sums_body(x_hbm, seg_hbm, s1_out, s2_out, cnt_out,
                  xbuf0, xbuf1, idxbuf0, idxbuf1, acc1, acc2, acc_c, dsem):
    c = lax.axis_index("c")
    s = lax.axis_index("s")
    wid = c * 16 + s

    zeros16 = jnp.zeros((16,), jnp.float32)
    ones16 = jnp.ones((16,), jnp.float32)

    def zero_body(g, carry):
        for j in range(NJ_K):
            acc1[g, pl.ds(j * 16, 16)] = zeros16
            acc2[g, pl.ds(j * 16, 16)] = zeros16
        acc_c[g, :] = zeros16
        return carry

    lax.fori_loop(0, G_K, zero_body, 0)

    n_ch = BASE_CHUNKS_K + jnp.where(wid < EXTRA_K, 1, 0)
    c0 = wid * BASE_CHUNKS_K + jnp.minimum(wid, EXTRA_K)

    def process(xbuf, idxbuf):
        s_first = jnp.max(plsc.load_gather(
            idxbuf, [jnp.zeros((16,), jnp.int32)]))
        s_last = jnp.max(plsc.load_gather(
            idxbuf, [jnp.full((16,), CHUNK_K - 1, jnp.int32)]))

        @pl.when(s_first == s_last)
        def _fast():
            def row_body(r2, accs):
                s1s, s2s = accs
                r = r2 * 2
                vs = [xbuf[r + rr, pl.ds(j * 16, 16)]
                      for rr in range(2) for j in range(NJ_K)]
                n1 = []
                n2 = []
                for j in range(NJ_K):
                    a, b = vs[j], vs[NJ_K + j]
                    n1.append(s1s[j] + (a + b))
                    n2.append(s2s[j] + (a * a + b * b))
                return (tuple(n1), tuple(n2))

            init = (tuple(zeros16 for _ in range(NJ_K)),
                    tuple(zeros16 for _ in range(NJ_K)))
            s1s, s2s = lax.fori_loop(0, CHUNK_K // 2, row_body, init)
            for j in range(NJ_K):
                sl = pl.ds(j * 16, 16)
                acc1[s_first, sl] = acc1[s_first, sl] + s1s[j]
                acc2[s_first, sl] = acc2[s_first, sl] + s2s[j]
            acc_c[s_first, :] = acc_c[s_first, :] + float(CHUNK_K) * ones16

        @pl.when(s_first != s_last)
        def _slow():
            def row_body(r, carry2):
                sv = jnp.max(plsc.load_gather(
                    idxbuf, [jnp.full((16,), r, jnp.int32)]))
                for j in range(NJ_K):
                    sl = pl.ds(j * 16, 16)
                    v = xbuf[r, sl]
                    acc1[sv, sl] = acc1[sv, sl] + v
                    acc2[sv, sl] = acc2[sv, sl] + v * v
                acc_c[sv, :] = acc_c[sv, :] + ones16
                return carry2

            lax.fori_loop(0, CHUNK_K, row_body, 0)

    pltpu.sync_copy(x_hbm.at[pl.ds(c0 * CHUNK_K, CHUNK_K)], xbuf0)
    pltpu.sync_copy(seg_hbm.at[pl.ds(c0 * CHUNK_K, CHUNK_K)], idxbuf0)

    def chunk_body(ch, carry):
        par = lax.rem(ch, 2)
        have_next = ch + 1 < n_ch
        base_next = (c0 + ch + 1) * CHUNK_K

        @pl.when(jnp.logical_and(have_next, par == 1))
        def _pf0():
            pltpu.async_copy(x_hbm.at[pl.ds(base_next, CHUNK_K)], xbuf0, dsem)
            pltpu.async_copy(seg_hbm.at[pl.ds(base_next, CHUNK_K)], idxbuf0, dsem)

        @pl.when(jnp.logical_and(have_next, par == 0))
        def _pf1():
            pltpu.async_copy(x_hbm.at[pl.ds(base_next, CHUNK_K)], xbuf1, dsem)
            pltpu.async_copy(seg_hbm.at[pl.ds(base_next, CHUNK_K)], idxbuf1, dsem)

        @pl.when(par == 0)
        def _p0():
            process(xbuf0, idxbuf0)

        @pl.when(par == 1)
        def _p1():
            process(xbuf1, idxbuf1)

        @pl.when(have_next)
        def _drain():
            pltpu.make_async_copy(
                x_hbm.at[pl.ds(base_next, CHUNK_K)], xbuf0, dsem).wait()
            pltpu.make_async_copy(
                seg_hbm.at[pl.ds(base_next, CHUNK_K)], idxbuf0, dsem).wait()

        return carry

    lax.fori_loop(0, n_ch, chunk_body, 0)

    pltpu.sync_copy(acc1, s1_out.at[wid])
    pltpu.sync_copy(acc2, s2_out.at[wid])
    pltpu.sync_copy(acc_c, cnt_out.at[wid])


def _sc_segment_sums(x, seg):
    mesh = plsc.VectorSubcoreMesh(core_axis_name="c", subcore_axis_name="s")
    fn = pl.kernel(
        _sc_sums_body,
        mesh=mesh,
        compiler_params=pltpu.CompilerParams(needs_layout_passes=False),
        out_type=[
            jax.ShapeDtypeStruct((NW_K, G_K, D_K), jnp.float32),
            jax.ShapeDtypeStruct((NW_K, G_K, D_K), jnp.float32),
            jax.ShapeDtypeStruct((NW_K, G_K, 16), jnp.float32),
        ],
        scratch_types=[
            pltpu.VMEM((CHUNK_K, D_K), jnp.float32),
            pltpu.VMEM((CHUNK_K, D_K), jnp.float32),
            pltpu.VMEM((CHUNK_K,), jnp.int32),
            pltpu.VMEM((CHUNK_K,), jnp.int32),
            pltpu.VMEM((G_K, D_K), jnp.float32),
            pltpu.VMEM((G_K, D_K), jnp.float32),
            pltpu.VMEM((G_K, 16), jnp.float32),
            pltpu.SemaphoreType.DMA,
        ],
    )
    return fn(x, seg)


def _tc_sums_body(x_ref, seg_ref, s1_ref, s2_ref, cnt_ref):
    i = pl.program_id(0)

    @pl.when(i == 0)
    def _init():
        s1_ref[...] = jnp.zeros_like(s1_ref)
        s2_ref[...] = jnp.zeros_like(s2_ref)
        cnt_ref[...] = jnp.zeros_like(cnt_ref)

    x = x_ref[...]
    seg = seg_ref[pl.ds(i + TC_BLK0, 1), :][0]
    oh_t = (lax.broadcasted_iota(jnp.int32, (G_K, ROWS_BLK), 0) == seg[None, :]).astype(jnp.float32)
    s1_ref[...] += jnp.dot(oh_t, x, preferred_element_type=jnp.float32)
    s2_ref[...] += jnp.dot(oh_t, x * x, preferred_element_type=jnp.float32)
    cnt_ref[...] += jnp.broadcast_to(jnp.sum(oh_t, axis=1, keepdims=True), (G_K, 128))


def _stats_body(s1p_ref, s2p_ref, cntp_ref, s1t_ref, s2t_ref, cntt_ref, b_ref):
    s1 = jnp.sum(s1p_ref[...], axis=0) + s1t_ref[...]
    s2 = jnp.sum(s2p_ref[...], axis=0) + s2t_ref[...]
    cnt = jnp.sum(cntp_ref[...], axis=0)[:, 0:1] + cntt_ref[:, 0:1]
    cnt = jnp.maximum(cnt, 1.0)
    inv = 1.0 / cnt
    mu = s1 * inv
    var = s2 * inv - mu * mu
    b_ref[...] = mu * lax.rsqrt(var + EPS_K)


def _apply_body(x_ref, seg_ref, b_ref, out_ref):
    seg = seg_ref[pl.ds(pl.program_id(0), 1), :][0]
    oh = (seg[:, None] == lax.broadcasted_iota(jnp.int32, (ROWS_BLK, G_K), 1)).astype(jnp.float32)
    out_ref[...] = x_ref[...] - jnp.dot(oh, b_ref[...], preferred_element_type=jnp.float32)


def kernel(x, segment_ids):
    seg = segment_ids.astype(jnp.int32)

    s1p, s2p, cntp = _sc_segment_sums(x, seg)
    seg2d = seg.reshape(N_BLKS, ROWS_BLK)

    s1t, s2t, cntt = pl.pallas_call(
        _tc_sums_body,
        grid=(N_TC_BLKS,),
        in_specs=[
            pl.BlockSpec((ROWS_BLK, D_K), lambda i: (i + TC_BLK0, 0)),
            pl.BlockSpec((N_BLKS, ROWS_BLK), lambda i: (0, 0)),
        ],
        out_specs=[
            pl.BlockSpec((G_K, D_K), lambda i: (0, 0)),
            pl.BlockSpec((G_K, D_K), lambda i: (0, 0)),
            pl.BlockSpec((G_K, 128), lambda i: (0, 0)),
        ],
        out_shape=[
            jax.ShapeDtypeStruct((G_K, D_K), jnp.float32),
            jax.ShapeDtypeStruct((G_K, D_K), jnp.float32),
            jax.ShapeDtypeStruct((G_K, 128), jnp.float32),
        ],
    )(x, seg2d)

    b = pl.pallas_call(
        _stats_body,
        out_shape=jax.ShapeDtypeStruct((G_K, D_K), jnp.float32),
    )(s1p, s2p, cntp, s1t, s2t, cntt)

    out = pl.pallas_call(
        _apply_body,
        grid=(N_BLKS,),
        in_specs=[
            pl.BlockSpec((ROWS_BLK, D_K), lambda i: (i, 0)),
            pl.BlockSpec((N_BLKS, ROWS_BLK), lambda i: (0, 0)),
            pl.BlockSpec((G_K, D_K), lambda i: (0, 0)),
        ],
        out_specs=pl.BlockSpec((ROWS_BLK, D_K), lambda i: (i, 0)),
        out_shape=jax.ShapeDtypeStruct((N_NODES_K, D_K), jnp.float32),
    )(x, seg2d, b)

    return out
